# baseline (device time: 7854 ns/iter reference)
import jax
import jax.numpy as jnp
from jax import lax
from jax.experimental import pallas as pl
from jax.experimental.pallas import tpu as pltpu

N_CHUNKS = 4


def kernel(x):
    m, n = x.shape
    rows = m // N_CHUNKS

    def body(x_hbm, out_ref, xbuf, acc_ref, comm_ref, copy_sems,
             send_sem, recv_sem):
        my_x = lax.axis_index("x")
        my_y = lax.axis_index("y")
        peer = (1 - my_x, my_y)

        copies = []
        for c in range(N_CHUNKS):
            cp = pltpu.make_async_copy(
                x_hbm.at[pl.ds(c * rows, rows), :],
                xbuf.at[c],
                copy_sems.at[c],
            )
            cp.start()
            copies.append(cp)

        barrier_sem = pltpu.get_barrier_semaphore()
        pl.semaphore_signal(
            barrier_sem, inc=1,
            device_id=peer, device_id_type=pl.DeviceIdType.MESH,
        )

        copies[0].wait()
        acc = jnp.sum(xbuf[0], axis=0, keepdims=True)
        for c in range(1, N_CHUNKS):
            copies[c].wait()
            acc = acc + jnp.sum(xbuf[c], axis=0, keepdims=True)
        acc_ref[:, :] = acc

        pl.semaphore_wait(barrier_sem, 1)

        rdma = pltpu.make_async_remote_copy(
            src_ref=acc_ref,
            dst_ref=comm_ref,
            send_sem=send_sem,
            recv_sem=recv_sem,
            device_id=peer,
            device_id_type=pl.DeviceIdType.MESH,
        )
        rdma.start()
        rdma.wait()

        out_ref[:, :] = acc_ref[:, :] + comm_ref[:, :]

    return pl.pallas_call(
        body,
        out_shape=jax.ShapeDtypeStruct((1, n), jnp.float32),
        in_specs=[pl.BlockSpec(memory_space=pl.ANY)],
        out_specs=pl.BlockSpec(memory_space=pltpu.VMEM),
        scratch_shapes=[
            pltpu.VMEM((N_CHUNKS, rows, n), jnp.float32),
            pltpu.VMEM((1, n), jnp.float32),
            pltpu.VMEM((1, n), jnp.float32),
            pltpu.SemaphoreType.DMA((N_CHUNKS,)),
            pltpu.SemaphoreType.DMA,
            pltpu.SemaphoreType.DMA,
        ],
        compiler_params=pltpu.CompilerParams(collective_id=0),
    )(x)


# device time: 6731 ns/iter; 1.1668x vs baseline; 1.1668x over previous
import jax
import jax.numpy as jnp
from jax import lax
from jax.experimental import pallas as pl
from jax.experimental.pallas import tpu as pltpu


def kernel(x):
    m, n = x.shape
    mh = m // 2
    nh = n // 2

    def body(x_hbm, out_ref, xbuf, acc_ref, comm_ref, out_vmem,
             copy_sems, out_copy_sem, send_sems, recv_sems):
        my_x = lax.axis_index("x")
        my_y = lax.axis_index("y")
        peer = (1 - my_x, my_y)

        copies = []
        for ch in range(2):
            for rh in range(2):
                cp = pltpu.make_async_copy(
                    x_hbm.at[pl.ds(rh * mh, mh), pl.ds(ch * nh, nh)],
                    xbuf.at[2 * ch + rh],
                    copy_sems.at[2 * ch + rh],
                )
                cp.start()
                copies.append(cp)

        barrier_sem = pltpu.get_barrier_semaphore()
        pl.semaphore_signal(
            barrier_sem, inc=1,
            device_id=peer, device_id_type=pl.DeviceIdType.MESH,
        )
        barrier_done = False

        rdmas = []
        for ch in range(2):
            copies[2 * ch].wait()
            part = jnp.sum(xbuf[2 * ch], axis=0, keepdims=True)
            copies[2 * ch + 1].wait()
            part = part + jnp.sum(xbuf[2 * ch + 1], axis=0, keepdims=True)
            acc_ref[:, pl.ds(ch * nh, nh)] = part
            if not barrier_done:
                pl.semaphore_wait(barrier_sem, 1)
                barrier_done = True
            rdma = pltpu.make_async_remote_copy(
                src_ref=acc_ref.at[:, pl.ds(ch * nh, nh)],
                dst_ref=comm_ref.at[:, pl.ds(ch * nh, nh)],
                send_sem=send_sems.at[ch],
                recv_sem=recv_sems.at[ch],
                device_id=peer,
                device_id_type=pl.DeviceIdType.MESH,
            )
            rdma.start()
            rdmas.append(rdma)

        for ch in range(2):
            rdmas[ch].wait_recv()
            out_vmem[:, pl.ds(ch * nh, nh)] = (
                acc_ref[:, pl.ds(ch * nh, nh)] + comm_ref[:, pl.ds(ch * nh, nh)]
            )
        out_cp = pltpu.make_async_copy(out_vmem, out_ref, out_copy_sem)
        out_cp.start()
        rdmas[0].wait_send()
        rdmas[1].wait_send()
        out_cp.wait()

    x = pltpu.with_memory_space_constraint(x, pltpu.MemorySpace.HBM)
    return pl.pallas_call(
        body,
        out_shape=jax.ShapeDtypeStruct((1, n), jnp.float32),
        in_specs=[pl.BlockSpec(memory_space=pltpu.MemorySpace.HBM)],
        out_specs=pl.BlockSpec(memory_space=pltpu.MemorySpace.HBM),
        scratch_shapes=[
            pltpu.VMEM((4, mh, nh), jnp.float32),
            pltpu.VMEM((1, n), jnp.float32),
            pltpu.VMEM((1, n), jnp.float32),
            pltpu.VMEM((1, n), jnp.float32),
            pltpu.SemaphoreType.DMA((4,)),
            pltpu.SemaphoreType.DMA,
            pltpu.SemaphoreType.DMA((2,)),
            pltpu.SemaphoreType.DMA((2,)),
        ],
        compiler_params=pltpu.CompilerParams(collective_id=0),
    )(x)


# device time: 6722 ns/iter; 1.1684x vs baseline; 1.0013x over previous
import jax
import jax.numpy as jnp
from jax import lax
from jax.experimental import pallas as pl
from jax.experimental.pallas import tpu as pltpu

N_CHUNKS = 3


def kernel(x):
    m, n = x.shape
    rows = m // N_CHUNKS

    def body(x_hbm, out_ref, xbuf, acc_ref, comm_ref, out_vmem,
             copy_sems, out_copy_sem, send_sem, recv_sem):
        my_x = lax.axis_index("x")
        my_y = lax.axis_index("y")
        peer = (1 - my_x, my_y)

        copies = []
        for c in range(N_CHUNKS):
            cp = pltpu.make_async_copy(
                x_hbm.at[pl.ds(c * rows, rows), :],
                xbuf.at[c],
                copy_sems.at[c],
            )
            cp.start()
            copies.append(cp)

        barrier_sem = pltpu.get_barrier_semaphore()
        pl.semaphore_signal(
            barrier_sem, inc=1,
            device_id=peer, device_id_type=pl.DeviceIdType.MESH,
        )

        copies[0].wait()
        acc = jnp.sum(xbuf[0], axis=0, keepdims=True)
        for c in range(1, N_CHUNKS):
            copies[c].wait()
            acc = acc + jnp.sum(xbuf[c], axis=0, keepdims=True)
        acc_ref[:, :] = acc

        pl.semaphore_wait(barrier_sem, 1)

        rdma = pltpu.make_async_remote_copy(
            src_ref=acc_ref,
            dst_ref=comm_ref,
            send_sem=send_sem,
            recv_sem=recv_sem,
            device_id=peer,
            device_id_type=pl.DeviceIdType.MESH,
        )
        rdma.start()
        rdma.wait_recv()
        out_vmem[:, :] = acc_ref[:, :] + comm_ref[:, :]
        out_cp = pltpu.make_async_copy(out_vmem, out_ref, out_copy_sem)
        out_cp.start()
        rdma.wait_send()
        out_cp.wait()

    x = pltpu.with_memory_space_constraint(x, pltpu.MemorySpace.HBM)
    return pl.pallas_call(
        body,
        out_shape=jax.ShapeDtypeStruct((1, n), jnp.float32),
        in_specs=[pl.BlockSpec(memory_space=pltpu.MemorySpace.HBM)],
        out_specs=pl.BlockSpec(memory_space=pl.ANY),
        scratch_shapes=[
            pltpu.VMEM((N_CHUNKS, rows, n), jnp.float32),
            pltpu.VMEM((1, n), jnp.float32),
            pltpu.VMEM((1, n), jnp.float32),
            pltpu.VMEM((1, n), jnp.float32),
            pltpu.SemaphoreType.DMA((N_CHUNKS,)),
            pltpu.SemaphoreType.DMA,
            pltpu.SemaphoreType.DMA,
            pltpu.SemaphoreType.DMA,
        ],
        compiler_params=pltpu.CompilerParams(collective_id=0),
    )(x)
